# baseline (speedup 1.0000x reference)
import jax
import jax.numpy as jnp
from jax import lax
from jax.experimental import pallas as pl
from jax.experimental.pallas import tpu as pltpu

B, C, H, W = 2, 96, 384, 384
NPIX = H * W
FAC = 6
HC = H // FAC
NCOARSE = HC * HC
SPLITM = 32
NBLK = SPLITM * SPLITM
SFINE = 12
NQ = SFINE * SFINE
TILE = FAC * FAC
G = 8
INV_SC2 = 1.0 / (96.0 ** 0.5)
PREC = lax.Precision.HIGHEST


def _dot(a, b):
    return jnp.dot(a, b, preferred_element_type=jnp.float32, precision=PREC)


def _dot_t(a, b):
    return lax.dot_general(a, b, (((1,), (1,)), ((), ())),
                           preferred_element_type=jnp.float32, precision=PREC)


def _mm_body(a_ref, b_ref, o_ref):
    o_ref[...] = _dot(a_ref[...], b_ref[...])


def _mm(a, bmat, rb):
    m, k = a.shape
    _, n = bmat.shape
    return pl.pallas_call(
        _mm_body,
        grid=(m // rb,),
        in_specs=[pl.BlockSpec((rb, k), lambda i: (i, 0)),
                  pl.BlockSpec((k, n), lambda i: (0, 0))],
        out_specs=pl.BlockSpec((rb, n), lambda i: (i, 0)),
        out_shape=jax.ShapeDtypeStruct((m, n), jnp.float32),
    )(a, bmat)


def _cproj_body(pq_ref, wqt_ref, bq_ref, pk_ref, wkt_ref, bk_ref,
                qd_ref, kd_ref):
    qd_ref[0] = _dot(pq_ref[0], wqt_ref[...]) + bq_ref[...]
    kd_ref[0] = _dot(pk_ref[0], wkt_ref[...]) + bk_ref[...]


def _topk_body(qd_ref, kd_ref, pair_ref, o_ref):
    q = qd_ref[0]
    k = kd_ref[0]
    e = (2.0 * _dot_t(q, k) - jnp.sum(k * k, axis=-1)[None, :]) * INV_SC2
    m = jnp.max(e, axis=-1, keepdims=True)
    p = jnp.exp(e - m)
    a = p / jnp.sum(p, axis=-1, keepdims=True)
    srow = a[0:HC, :] + a[HC:2 * HC, :]
    s = _dot(pair_ref[...], srow)
    idx = lax.broadcasted_iota(jnp.int32, s.shape, 1)
    m1 = jnp.max(s, axis=-1, keepdims=True)
    i1 = jnp.min(jnp.where(s == m1, idx, NCOARSE), axis=-1, keepdims=True)
    s2 = jnp.where(idx == i1, -jnp.inf, s)
    m2 = jnp.max(s2, axis=-1, keepdims=True)
    i2 = jnp.min(jnp.where(s2 == m2, idx, NCOARSE), axis=-1, keepdims=True)
    o_ref[0] = jnp.concatenate([i1, i2], axis=-1)


def _phase2_body(tk_ref, xq_ref, *rest):
    del tk_ref
    tiles = rest[:2 * G]
    (wqt_ref, bq_ref, wkvt_ref, bkv_ref,
     wp1t_ref, wp2t_ref, bp_ref, o_ref) = rest[2 * G:]
    xq = xq_ref[0]
    q = _dot(xq, wqt_ref[...]) + bq_ref[...]
    t_all = jnp.concatenate([t[0, 0] for t in tiles], axis=0)
    kv = _dot(t_all, wkvt_ref[...]) + bkv_ref[...]
    k, v = kv[:, 0:C], kv[:, C:2 * C]
    e = (2.0 * _dot_t(q, k) - jnp.sum(k * k, axis=-1)[None, :]) * INV_SC2
    rows = lax.broadcasted_iota(jnp.int32, e.shape, 0) // NQ
    cols = lax.broadcasted_iota(jnp.int32, e.shape, 1) // (2 * TILE)
    e = jnp.where(rows == cols, e, -jnp.inf)
    m = jnp.max(e, axis=-1, keepdims=True)
    p = jnp.exp(e - m)
    a = p / jnp.sum(p, axis=-1, keepdims=True)
    att = _dot(a, v)
    o_ref[0] = (_dot(xq, wp1t_ref[...]) + _dot(att, wp2t_ref[...])
                + bp_ref[...])


def _pool_img(x, pt):
    a = x.reshape(B * C * H, W)
    c1 = _mm(a, pt, 1024)
    c1 = c1.reshape(B, C, H, HC).transpose(0, 1, 3, 2).reshape(B * C * HC, H)
    c2 = _mm(c1, pt, 1024)
    return c2.reshape(B, C, HC, HC).transpose(0, 3, 2, 1).reshape(B, NCOARSE, C)


def kernel(xKeyValue, xQuery, W_val, b_val, W_key, b_key, W_q, b_q,
           W_proj, b_proj):
    f32 = jnp.float32
    pt = (jnp.arange(W)[:, None] // FAC == jnp.arange(HC)[None, :]).astype(f32) / FAC
    pair = (jnp.arange(SPLITM)[:, None] == jnp.arange(HC)[None, :] // 2).astype(f32)

    xkv_r = (xKeyValue.reshape(B, C, HC, FAC, HC, FAC)
             .transpose(0, 2, 4, 3, 5, 1).reshape(B, NCOARSE, TILE, C))
    xq_r = (xQuery.reshape(B, C, SPLITM, SFINE, SPLITM, SFINE)
            .transpose(0, 2, 4, 3, 5, 1).reshape(B, NPIX, C))

    pq_t = _pool_img(xQuery, pt)
    pkv_t = _pool_img(xKeyValue, pt)

    qd, kd = pl.pallas_call(
        _cproj_body,
        grid=(B,),
        in_specs=[pl.BlockSpec((1, NCOARSE, C), lambda b: (b, 0, 0)),
                  pl.BlockSpec((C, C), lambda b: (0, 0)),
                  pl.BlockSpec((1, C), lambda b: (0, 0)),
                  pl.BlockSpec((1, NCOARSE, C), lambda b: (b, 0, 0)),
                  pl.BlockSpec((C, C), lambda b: (0, 0)),
                  pl.BlockSpec((1, C), lambda b: (0, 0))],
        out_specs=[pl.BlockSpec((1, NCOARSE, C), lambda b: (b, 0, 0)),
                   pl.BlockSpec((1, NCOARSE, C), lambda b: (b, 0, 0))],
        out_shape=[jax.ShapeDtypeStruct((B, NCOARSE, C), f32),
                   jax.ShapeDtypeStruct((B, NCOARSE, C), f32)],
    )(pq_t, W_q.T, b_q[None, :], pkv_t, W_key.T, b_key[None, :])

    tk = pl.pallas_call(
        _topk_body,
        grid=(B, SPLITM),
        in_specs=[pl.BlockSpec((1, 2 * HC, C), lambda b, i: (b, i, 0)),
                  pl.BlockSpec((1, NCOARSE, C), lambda b, i: (b, 0, 0)),
                  pl.BlockSpec((SPLITM, HC), lambda b, i: (0, 0))],
        out_specs=pl.BlockSpec((1, SPLITM, 2), lambda b, i: (b, i, 0)),
        out_shape=jax.ShapeDtypeStruct((B, NBLK, 2), jnp.int32),
    )(qd, kd, pair)

    def _tile_spec(i, kap):
        def imap(b, j, tkr, i=i, kap=kap):
            return (b, tkr[(b * NBLK + G * j + i) * 2 + kap], 0, 0)
        return pl.BlockSpec((1, 1, TILE, C), imap)

    tile_specs = []
    for i in range(G):
        tile_specs.append(_tile_spec(i, 0))
        tile_specs.append(_tile_spec(i, 1))

    grid_spec = pltpu.PrefetchScalarGridSpec(
        num_scalar_prefetch=1,
        grid=(B, NBLK // G),
        in_specs=[pl.BlockSpec((1, G * NQ, C), lambda b, j, tkr: (b, j, 0))]
                 + tile_specs
                 + [pl.BlockSpec((C, C), lambda b, j, tkr: (0, 0)),
                    pl.BlockSpec((1, C), lambda b, j, tkr: (0, 0)),
                    pl.BlockSpec((C, 2 * C), lambda b, j, tkr: (0, 0)),
                    pl.BlockSpec((1, 2 * C), lambda b, j, tkr: (0, 0)),
                    pl.BlockSpec((C, C), lambda b, j, tkr: (0, 0)),
                    pl.BlockSpec((C, C), lambda b, j, tkr: (0, 0)),
                    pl.BlockSpec((1, C), lambda b, j, tkr: (0, 0))],
        out_specs=pl.BlockSpec((1, G * NQ, C), lambda b, j, tkr: (b, j, 0)),
    )
    out_r = pl.pallas_call(
        _phase2_body,
        grid_spec=grid_spec,
        out_shape=jax.ShapeDtypeStruct((B, NPIX, C), f32),
    )(tk.reshape(-1), xq_r, *([xkv_r] * (2 * G)),
      W_q.T, b_q[None, :],
      jnp.concatenate([W_key.T, W_val.T], axis=1),
      jnp.concatenate([b_key, b_val])[None, :],
      W_proj[:, 0:C].T, W_proj[:, C:2 * C].T, b_proj[None, :])

    out = (out_r.reshape(B, SPLITM, SPLITM, SFINE, SFINE, C)
           .transpose(0, 5, 1, 3, 2, 4).reshape(B, C, H, W))
    return out


# speedup vs baseline: 1.3998x; 1.3998x over previous
import jax
import jax.numpy as jnp
from jax import lax
from jax.experimental import pallas as pl
from jax.experimental.pallas import tpu as pltpu

B, C, H, W = 2, 96, 384, 384
NPIX = H * W
FAC = 6
HC = H // FAC
NCOARSE = HC * HC
SPLITM = 32
NBLK = SPLITM * SPLITM
SFINE = 12
NQ = SFINE * SFINE
TILE = FAC * FAC
G = 8
INV_SC2 = 1.0 / (96.0 ** 0.5)
PREC = lax.Precision.HIGHEST


def _dot(a, b):
    return jnp.dot(a, b, preferred_element_type=jnp.float32, precision=PREC)


def _dot_t(a, b):
    return lax.dot_general(a, b, (((1,), (1,)), ((), ())),
                           preferred_element_type=jnp.float32, precision=PREC)


def _bdot_t(a, b):
    return lax.dot_general(a, b, (((2,), (2,)), ((0,), (0,))),
                           preferred_element_type=jnp.float32, precision=PREC)


def _bdot(a, b):
    return lax.dot_general(a, b, (((2,), (1,)), ((0,), (0,))),
                           preferred_element_type=jnp.float32, precision=PREC)


def _mm_body(a_ref, b_ref, o_ref):
    o_ref[...] = _dot(a_ref[...], b_ref[...])


def _mm(a, bmat, rb):
    m, k = a.shape
    _, n = bmat.shape
    return pl.pallas_call(
        _mm_body,
        grid=(m // rb,),
        in_specs=[pl.BlockSpec((rb, k), lambda i: (i, 0)),
                  pl.BlockSpec((k, n), lambda i: (0, 0))],
        out_specs=pl.BlockSpec((rb, n), lambda i: (i, 0)),
        out_shape=jax.ShapeDtypeStruct((m, n), jnp.float32),
    )(a, bmat)


def _cproj_body(pq_ref, wqt_ref, bq_ref, pk_ref, wkt_ref, bk_ref,
                qd_ref, kd_ref):
    qd_ref[0] = _dot(pq_ref[0], wqt_ref[...]) + bq_ref[...]
    kd_ref[0] = _dot(pk_ref[0], wkt_ref[...]) + bk_ref[...]


def _topk_body(qd_ref, kd_ref, pair_ref, o_ref):
    q = qd_ref[0]
    k = kd_ref[0]
    e = (2.0 * _dot_t(q, k) - jnp.sum(k * k, axis=-1)[None, :]) * INV_SC2
    m = jnp.max(e, axis=-1, keepdims=True)
    p = jnp.exp(e - m)
    a = p / jnp.sum(p, axis=-1, keepdims=True)
    srow = a[0:HC, :] + a[HC:2 * HC, :]
    s = _dot(pair_ref[...], srow)
    idx = lax.broadcasted_iota(jnp.int32, s.shape, 1)
    m1 = jnp.max(s, axis=-1, keepdims=True)
    i1 = jnp.min(jnp.where(s == m1, idx, NCOARSE), axis=-1, keepdims=True)
    s2 = jnp.where(idx == i1, -jnp.inf, s)
    m2 = jnp.max(s2, axis=-1, keepdims=True)
    i2 = jnp.min(jnp.where(s2 == m2, idx, NCOARSE), axis=-1, keepdims=True)
    o_ref[0] = jnp.concatenate([i1, i2], axis=-1)


def _phase2_body(tk_ref, xq_ref, *rest):
    del tk_ref
    tiles = rest[:2 * G]
    (wqt_ref, bq_ref, wkvt_ref, bkv_ref,
     wp1t_ref, wp2t_ref, bp_ref, o_ref) = rest[2 * G:]
    xq = xq_ref[0]
    q = _dot(xq, wqt_ref[...]) + bq_ref[...]
    t_all = jnp.concatenate([t[0, 0] for t in tiles], axis=0)
    kv = _dot(t_all, wkvt_ref[...]) + bkv_ref[...]
    k, v = kv[:, 0:C], kv[:, C:2 * C]
    qb = q.reshape(G, NQ, C)
    kb = k.reshape(G, 2 * TILE, C)
    vb = v.reshape(G, 2 * TILE, C)
    e = (2.0 * _bdot_t(qb, kb)
         - jnp.sum(kb * kb, axis=-1)[:, None, :]) * INV_SC2
    m = jnp.max(e, axis=-1, keepdims=True)
    p = jnp.exp(e - m)
    a = p / jnp.sum(p, axis=-1, keepdims=True)
    att = _bdot(a, vb).reshape(G * NQ, C)
    o_ref[0] = (_dot(xq, wp1t_ref[...]) + _dot(att, wp2t_ref[...])
                + bp_ref[...])


def _pool_img(x, pt):
    a = x.reshape(B * C * H, W)
    c1 = _mm(a, pt, 1024)
    c1 = c1.reshape(B, C, H, HC).transpose(0, 1, 3, 2).reshape(B * C * HC, H)
    c2 = _mm(c1, pt, 1024)
    return c2.reshape(B, C, HC, HC).transpose(0, 3, 2, 1).reshape(B, NCOARSE, C)


def kernel(xKeyValue, xQuery, W_val, b_val, W_key, b_key, W_q, b_q,
           W_proj, b_proj):
    f32 = jnp.float32
    pt = (jnp.arange(W)[:, None] // FAC == jnp.arange(HC)[None, :]).astype(f32) / FAC
    pair = (jnp.arange(SPLITM)[:, None] == jnp.arange(HC)[None, :] // 2).astype(f32)

    xkv_r = (xKeyValue.reshape(B, C, HC, FAC, HC, FAC)
             .transpose(0, 2, 4, 3, 5, 1).reshape(B, NCOARSE, TILE, C))
    xq_r = (xQuery.reshape(B, C, SPLITM, SFINE, SPLITM, SFINE)
            .transpose(0, 2, 4, 3, 5, 1).reshape(B, NPIX, C))

    pq_t = _pool_img(xQuery, pt)
    pkv_t = _pool_img(xKeyValue, pt)

    qd, kd = pl.pallas_call(
        _cproj_body,
        grid=(B,),
        in_specs=[pl.BlockSpec((1, NCOARSE, C), lambda b: (b, 0, 0)),
                  pl.BlockSpec((C, C), lambda b: (0, 0)),
                  pl.BlockSpec((1, C), lambda b: (0, 0)),
                  pl.BlockSpec((1, NCOARSE, C), lambda b: (b, 0, 0)),
                  pl.BlockSpec((C, C), lambda b: (0, 0)),
                  pl.BlockSpec((1, C), lambda b: (0, 0))],
        out_specs=[pl.BlockSpec((1, NCOARSE, C), lambda b: (b, 0, 0)),
                   pl.BlockSpec((1, NCOARSE, C), lambda b: (b, 0, 0))],
        out_shape=[jax.ShapeDtypeStruct((B, NCOARSE, C), f32),
                   jax.ShapeDtypeStruct((B, NCOARSE, C), f32)],
    )(pq_t, W_q.T, b_q[None, :], pkv_t, W_key.T, b_key[None, :])

    tk = pl.pallas_call(
        _topk_body,
        grid=(B, SPLITM),
        in_specs=[pl.BlockSpec((1, 2 * HC, C), lambda b, i: (b, i, 0)),
                  pl.BlockSpec((1, NCOARSE, C), lambda b, i: (b, 0, 0)),
                  pl.BlockSpec((SPLITM, HC), lambda b, i: (0, 0))],
        out_specs=pl.BlockSpec((1, SPLITM, 2), lambda b, i: (b, i, 0)),
        out_shape=jax.ShapeDtypeStruct((B, NBLK, 2), jnp.int32),
    )(qd, kd, pair)

    def _tile_spec(i, kap):
        def imap(b, j, tkr, i=i, kap=kap):
            return (b, tkr[(b * NBLK + G * j + i) * 2 + kap], 0, 0)
        return pl.BlockSpec((1, 1, TILE, C), imap)

    tile_specs = []
    for i in range(G):
        tile_specs.append(_tile_spec(i, 0))
        tile_specs.append(_tile_spec(i, 1))

    grid_spec = pltpu.PrefetchScalarGridSpec(
        num_scalar_prefetch=1,
        grid=(B, NBLK // G),
        in_specs=[pl.BlockSpec((1, G * NQ, C), lambda b, j, tkr: (b, j, 0))]
                 + tile_specs
                 + [pl.BlockSpec((C, C), lambda b, j, tkr: (0, 0)),
                    pl.BlockSpec((1, C), lambda b, j, tkr: (0, 0)),
                    pl.BlockSpec((C, 2 * C), lambda b, j, tkr: (0, 0)),
                    pl.BlockSpec((1, 2 * C), lambda b, j, tkr: (0, 0)),
                    pl.BlockSpec((C, C), lambda b, j, tkr: (0, 0)),
                    pl.BlockSpec((C, C), lambda b, j, tkr: (0, 0)),
                    pl.BlockSpec((1, C), lambda b, j, tkr: (0, 0))],
        out_specs=pl.BlockSpec((1, G * NQ, C), lambda b, j, tkr: (b, j, 0)),
    )
    out_r = pl.pallas_call(
        _phase2_body,
        grid_spec=grid_spec,
        out_shape=jax.ShapeDtypeStruct((B, NPIX, C), f32),
    )(tk.reshape(-1), xq_r, *([xkv_r] * (2 * G)),
      W_q.T, b_q[None, :],
      jnp.concatenate([W_key.T, W_val.T], axis=1),
      jnp.concatenate([b_key, b_val])[None, :],
      W_proj[:, 0:C].T, W_proj[:, C:2 * C].T, b_proj[None, :])

    out = (out_r.reshape(B, SPLITM, SPLITM, SFINE, SFINE, C)
           .transpose(0, 5, 1, 3, 2, 4).reshape(B, C, H, W))
    return out
